# baseline (speedup 1.0000x reference)
import dataclasses
import functools

import jax
import jax.numpy as jnp
from jax import lax
from jax.experimental import pallas as pl
from jax.experimental.pallas import tpu as pltpu
from jax.experimental.pallas import tpu_sc as plsc

N = 10000
K = 32
D = 128
C = 10

NT = 32
CH = 4
ECH = CH * K
TPT = 80
NCH_PAD = NT * TPT
NPAD = NCH_PAD * CH
EPAD = NCH_PAD * ECH


def _sc_gather_reduce(src_tok, esrc_pad, emb):
    mesh = plsc.VectorSubcoreMesh(core_axis_name="c", subcore_axis_name="s")
    cp = pltpu.CompilerParams()
    if "needs_layout_passes" in pltpu.CompilerParams.__dataclass_fields__:
        cp = dataclasses.replace(cp, needs_layout_passes=False)

    @functools.partial(
        pl.kernel,
        compiler_params=cp,
        out_type=[
            jax.ShapeDtypeStruct((NPAD, D), jnp.float32),
            jax.ShapeDtypeStruct((NPAD, D), jnp.float32),
        ],
        mesh=mesh,
        scratch_types=[
            pltpu.VMEM((N,), jnp.int32),
            pltpu.VMEM((TPT * ECH,), jnp.int32),
            pltpu.VMEM((ECH,), jnp.int32),
            pltpu.VMEM((ECH,), jnp.int32),
            pltpu.VMEM((ECH, D), jnp.float32),
            pltpu.VMEM((ECH, D), jnp.float32),
            pltpu.VMEM((TPT * CH // 2, D), jnp.float32),
            pltpu.VMEM((TPT * CH // 2, D), jnp.float32),
            pltpu.SemaphoreType.DMA,
            pltpu.SemaphoreType.DMA,
        ],
    )
    def sc_kernel(tok_hbm, esrc_hbm, emb_hbm, total_hbm, xlast_hbm,
                  tok_v, eidx_v, tokidx0, tokidx1, rows0, rows1,
                  stot, sxl, semg0, semg1):
        wid = lax.axis_index("s") * 2 + lax.axis_index("c")
        base_c = wid * TPT
        pltpu.sync_copy(tok_hbm, tok_v)
        pltpu.sync_copy(esrc_hbm.at[pl.ds(base_c * ECH, TPT * ECH)], eidx_v)

        def compose_and_gather(x, tokidx, rows, semg):
            for i8 in range(ECH // 16):
                sl = pl.ds(i8 * 16, 16)
                tokidx[sl] = plsc.load_gather(
                    tok_v, [eidx_v[pl.ds(x * ECH + i8 * 16, 16)]])
            pltpu.async_copy(emb_hbm.at[tokidx], rows, semg)

        HALF = TPT // 2

        def process(x, tokidx, rows, semg):
            srow = jnp.where(x >= HALF, x - HALF, x) * CH
            pltpu.make_async_copy(emb_hbm.at[tokidx], rows, semg).wait()
            for j in range(CH):
                for cb in range(D // 16):
                    sl = pl.ds(cb * 16, 16)
                    acc = rows[j * K, sl]
                    for k in range(1, K):
                        acc = acc + rows[j * K + k, sl]
                    stot[srow + j, sl] = acc
                    sxl[srow + j, sl] = rows[j * K + K - 1, sl]

        compose_and_gather(0, tokidx0, rows0, semg0)

        def body(i, carry):
            a = 2 * i
            compose_and_gather(a + 1, tokidx1, rows1, semg1)
            process(a, tokidx0, rows0, semg0)

            @pl.when(i < TPT // 2 - 1)
            def _():
                compose_and_gather(a + 2, tokidx0, rows0, semg0)

            process(a + 1, tokidx1, rows1, semg1)

            @pl.when(i == HALF // 2 - 1)
            def _():
                pltpu.sync_copy(
                    stot, total_hbm.at[pl.ds(base_c * CH, HALF * CH)])
                pltpu.sync_copy(
                    sxl, xlast_hbm.at[pl.ds(base_c * CH, HALF * CH)])

            return carry

        lax.fori_loop(0, TPT // 2, body, 0)

        mid = (base_c + HALF) * CH
        pltpu.sync_copy(stot, total_hbm.at[pl.ds(mid, HALF * CH)])
        pltpu.sync_copy(sxl, xlast_hbm.at[pl.ds(mid, HALF * CH)])

    return sc_kernel(src_tok, esrc_pad, emb)


def _tc_body(x_ref, tot_ref, wih_ref, whh_ref, bih_ref, bhh_ref,
             lng_ref, lnb_ref, wfc_ref, bfc_ref, o_ref):
    x = x_ref[...]
    h = tot_ref[...] - x
    gi = jnp.dot(x, wih_ref[...], preferred_element_type=jnp.float32) + bih_ref[...]
    gh = jnp.dot(h, whh_ref[...], preferred_element_type=jnp.float32) + bhh_ref[...]
    r = jax.nn.sigmoid(gi[:, :D] + gh[:, :D])
    z = jax.nn.sigmoid(gi[:, D:2 * D] + gh[:, D:2 * D])
    n = jnp.tanh(gi[:, 2 * D:] + r * gh[:, 2 * D:])
    ho = (1.0 - z) * n + z * h
    mu = jnp.mean(ho, axis=-1, keepdims=True)
    var = jnp.mean((ho - mu) ** 2, axis=-1, keepdims=True)
    rst = lng_ref[...] * (ho - mu) / jnp.sqrt(var + 1e-5) + lnb_ref[...]
    o_ref[...] = jnp.dot(rst, wfc_ref[...], preferred_element_type=jnp.float32) + bfc_ref[...]


def _tc_head(x, total, W_ihT, W_hhT, b_ih2, b_hh2, ln_g2, ln_b2, W_fcT, b_fc2):
    B = 1024
    grid = (NPAD // B,)
    full = lambda shape: pl.BlockSpec(shape, lambda i: (0, 0))
    return pl.pallas_call(
        _tc_body,
        grid=grid,
        in_specs=[
            pl.BlockSpec((B, D), lambda i: (i, 0)),
            pl.BlockSpec((B, D), lambda i: (i, 0)),
            full((D, 3 * D)),
            full((D, 3 * D)),
            full((1, 3 * D)),
            full((1, 3 * D)),
            full((1, D)),
            full((1, D)),
            full((D, D)),
            full((1, D)),
        ],
        out_specs=pl.BlockSpec((B, D), lambda i: (i, 0)),
        out_shape=jax.ShapeDtypeStruct((NPAD, D), jnp.float32),
    )(x, total, W_ihT, W_hhT, b_ih2, b_hh2, ln_g2, ln_b2, W_fcT, b_fc2)


def kernel(src_token_ids, dst_token_ids, edge_index, emb, W_ih, W_hh,
           b_ih, b_hh, ln_g, ln_b, W_fc, b_fc):
    src_tok = src_token_ids.astype(jnp.int32)
    esrc_pad = jnp.pad(edge_index[0].astype(jnp.int32), (0, EPAD - N * K))
    total, xlast = _sc_gather_reduce(src_tok, esrc_pad, emb)

    W_ihT = W_ih.T
    W_hhT = W_hh.T
    b_ih2 = b_ih.reshape(1, 3 * D)
    b_hh2 = b_hh.reshape(1, 3 * D)
    ln_g2 = ln_g.reshape(1, D)
    ln_b2 = ln_b.reshape(1, D)
    W_fcT = jnp.pad(W_fc.T, ((0, 0), (0, D - C)))
    b_fc2 = jnp.pad(b_fc, (0, D - C)).reshape(1, D)

    out_pad = _tc_head(xlast, total, W_ihT, W_hhT, b_ih2, b_hh2,
                       ln_g2, ln_b2, W_fcT, b_fc2)
    return out_pad[:N, :C]


# speedup vs baseline: 1.7550x; 1.7550x over previous
import dataclasses
import functools

import jax
import jax.numpy as jnp
from jax import lax
from jax.experimental import pallas as pl
from jax.experimental.pallas import tpu as pltpu
from jax.experimental.pallas import tpu_sc as plsc

N = 10000
K = 32
D = 128
C = 10

NT = 32
NS = 16
CH = 4
ECH = CH * K
TPT = 80
NCH_PAD = NT * TPT
NPAD = NCH_PAD * CH
EPAD = NCH_PAD * ECH
RPT = NPAD // NS
GPH1 = RPT // ECH


def _sc_gather_reduce(src_tok_pad, esrc_pad, emb):
    mesh = plsc.VectorSubcoreMesh(core_axis_name="c", subcore_axis_name="s")
    cp = pltpu.CompilerParams()
    if "needs_layout_passes" in pltpu.CompilerParams.__dataclass_fields__:
        cp = dataclasses.replace(cp, needs_layout_passes=False)

    @functools.partial(
        pl.kernel,
        compiler_params=cp,
        out_type=[
            jax.ShapeDtypeStruct((NPAD, D), jnp.float32),
            jax.ShapeDtypeStruct((NPAD, D), jnp.float32),
        ],
        mesh=mesh,
        scratch_types=[
            pltpu.VMEM_SHARED((NPAD, D), jnp.float32),
            pltpu.VMEM((TPT * ECH,), jnp.int32),
            pltpu.VMEM((ECH,), jnp.int32),
            pltpu.VMEM((ECH, D), jnp.float32),
            pltpu.VMEM((ECH, D), jnp.float32),
            pltpu.VMEM((CH, D), jnp.float32),
            pltpu.VMEM((CH, D), jnp.float32),
            pltpu.VMEM((CH, D), jnp.float32),
            pltpu.VMEM((CH, D), jnp.float32),
            pltpu.SemaphoreType.DMA,
            pltpu.SemaphoreType.DMA,
            pltpu.SemaphoreType.DMA,
            pltpu.SemaphoreType.DMA,
            pltpu.SemaphoreType.DMA,
            pltpu.SemaphoreType.DMA,
        ],
    )
    def sc_kernel(tok_hbm, esrc_hbm, emb_hbm, total_hbm, xlast_hbm,
                  tab_sh, eidx_v, ridx_v, rows0, rows1, totb0, totb1, xlb0, xlb1,
                  semg0, semg1, semt0, semt1, semx0, semx1):
        cid = lax.axis_index("c")
        sid = lax.axis_index("s")

        pltpu.sync_copy(tok_hbm.at[pl.ds(sid * RPT, RPT)],
                        eidx_v.at[pl.ds(0, RPT)])
        for g in range(GPH1):
            pltpu.async_copy(
                emb_hbm.at[eidx_v.at[pl.ds(g * ECH, ECH)]], rows0, semg0
            ).wait()
            base_row = sid * RPT + g * ECH
            for i8 in range(ECH // 16):
                ridx_v[pl.ds(i8 * 16, 16)] = (
                    base_row + i8 * 16 + lax.iota(jnp.int32, 16))
            pltpu.sync_copy(rows0, tab_sh.at[ridx_v])
        plsc.subcore_barrier()

        wid = sid * 2 + cid
        base_c = wid * TPT
        pltpu.sync_copy(esrc_hbm.at[pl.ds(base_c * ECH, TPT * ECH)], eidx_v)

        def start_gather(x, rows, semg):
            pltpu.async_copy(
                tab_sh.at[eidx_v.at[pl.ds(x * ECH, ECH)]], rows, semg)

        def process(x, rows, semg, totb, xlb, semt, semx, not_first):
            row = (base_c + x) * CH

            @pl.when(not_first)
            def _():
                pltpu.make_async_copy(
                    totb, total_hbm.at[pl.ds(row, CH)], semt).wait()
                pltpu.make_async_copy(
                    xlb, xlast_hbm.at[pl.ds(row, CH)], semx).wait()

            pltpu.make_async_copy(
                tab_sh.at[eidx_v.at[pl.ds(x * ECH, ECH)]], rows, semg).wait()
            for j in range(CH):
                for cb in range(D // 16):
                    sl = pl.ds(cb * 16, 16)
                    totb[j, sl] = rows[j * K, sl]
                    xlb[j, sl] = rows[j * K + K - 1, sl]

            def kbody(k, kcarry):
                for j in range(CH):
                    for cb in range(D // 16):
                        sl = pl.ds(cb * 16, 16)
                        plsc.addupdate(totb.at[j, sl], rows[j * K + k, sl])
                return kcarry

            lax.fori_loop(1, K, kbody, 0)
            pltpu.async_copy(totb, total_hbm.at[pl.ds(row, CH)], semt)
            pltpu.async_copy(xlb, xlast_hbm.at[pl.ds(row, CH)], semx)

        start_gather(0, rows0, semg0)

        def body(i, carry):
            a = 2 * i
            start_gather(a + 1, rows1, semg1)
            process(a, rows0, semg0, totb0, xlb0, semt0, semx0,
                    not_first=i >= 1)

            @pl.when(i < TPT // 2 - 1)
            def _():
                start_gather(a + 2, rows0, semg0)

            process(a + 1, rows1, semg1, totb1, xlb1, semt1, semx1,
                    not_first=i >= 1)
            return carry

        lax.fori_loop(0, TPT // 2, body, 0)

        r0 = (base_c + TPT - 2) * CH
        r1 = (base_c + TPT - 1) * CH
        pltpu.make_async_copy(totb0, total_hbm.at[pl.ds(r0, CH)], semt0).wait()
        pltpu.make_async_copy(xlb0, xlast_hbm.at[pl.ds(r0, CH)], semx0).wait()
        pltpu.make_async_copy(totb1, total_hbm.at[pl.ds(r1, CH)], semt1).wait()
        pltpu.make_async_copy(xlb1, xlast_hbm.at[pl.ds(r1, CH)], semx1).wait()

    return sc_kernel(src_tok_pad, esrc_pad, emb)


def _tc_body(x_ref, tot_ref, wih_ref, whh_ref, bih_ref, bhh_ref,
             lng_ref, lnb_ref, wfc_ref, bfc_ref, o_ref):
    x = x_ref[...]
    h = tot_ref[...] - x
    gi = jnp.dot(x, wih_ref[...], preferred_element_type=jnp.float32) + bih_ref[...]
    gh = jnp.dot(h, whh_ref[...], preferred_element_type=jnp.float32) + bhh_ref[...]
    r = jax.nn.sigmoid(gi[:, :D] + gh[:, :D])
    z = jax.nn.sigmoid(gi[:, D:2 * D] + gh[:, D:2 * D])
    n = jnp.tanh(gi[:, 2 * D:] + r * gh[:, 2 * D:])
    ho = (1.0 - z) * n + z * h
    mu = jnp.mean(ho, axis=-1, keepdims=True)
    var = jnp.mean((ho - mu) ** 2, axis=-1, keepdims=True)
    rst = lng_ref[...] * (ho - mu) / jnp.sqrt(var + 1e-5) + lnb_ref[...]
    o_ref[...] = jnp.dot(rst, wfc_ref[...], preferred_element_type=jnp.float32) + bfc_ref[...]


def _tc_head(x, total, W_ihT, W_hhT, b_ih2, b_hh2, ln_g2, ln_b2, W_fcT, b_fc2):
    B = 1024
    grid = (NPAD // B,)
    full = lambda shape: pl.BlockSpec(shape, lambda i: (0, 0))
    return pl.pallas_call(
        _tc_body,
        grid=grid,
        in_specs=[
            pl.BlockSpec((B, D), lambda i: (i, 0)),
            pl.BlockSpec((B, D), lambda i: (i, 0)),
            full((D, 3 * D)),
            full((D, 3 * D)),
            full((1, 3 * D)),
            full((1, 3 * D)),
            full((1, D)),
            full((1, D)),
            full((D, D)),
            full((1, D)),
        ],
        out_specs=pl.BlockSpec((B, D), lambda i: (i, 0)),
        out_shape=jax.ShapeDtypeStruct((NPAD, D), jnp.float32),
    )(x, total, W_ihT, W_hhT, b_ih2, b_hh2, ln_g2, ln_b2, W_fcT, b_fc2)


def kernel(src_token_ids, dst_token_ids, edge_index, emb, W_ih, W_hh,
           b_ih, b_hh, ln_g, ln_b, W_fc, b_fc):
    src_tok_pad = jnp.pad(src_token_ids.astype(jnp.int32), (0, NPAD - N))
    esrc_pad = jnp.pad(edge_index[0].astype(jnp.int32), (0, EPAD - N * K))
    total, xlast = _sc_gather_reduce(src_tok_pad, esrc_pad, emb)

    W_ihT = W_ih.T
    W_hhT = W_hh.T
    b_ih2 = b_ih.reshape(1, 3 * D)
    b_hh2 = b_hh.reshape(1, 3 * D)
    ln_g2 = ln_g.reshape(1, D)
    ln_b2 = ln_b.reshape(1, D)
    W_fcT = jnp.pad(W_fc.T, ((0, 0), (0, D - C)))
    b_fc2 = jnp.pad(b_fc, (0, D - C)).reshape(1, D)

    out_pad = _tc_head(xlast, total, W_ihT, W_hhT, b_ih2, b_hh2,
                       ln_g2, ln_b2, W_fcT, b_fc2)
    return out_pad[:N, :C]


# speedup vs baseline: 4.3795x; 2.4954x over previous
import dataclasses
import functools

import jax
import jax.numpy as jnp
from jax import lax
from jax.experimental import pallas as pl
from jax.experimental.pallas import tpu as pltpu
from jax.experimental.pallas import tpu_sc as plsc

N = 10000
K = 32
D = 128
C = 10

NT = 32
NS = 16
CH = 4
ECH = CH * K
TPT = 80
NCH_PAD = NT * TPT
NPAD = NCH_PAD * CH
EPAD = NCH_PAD * ECH
RPT = NPAD // NS
GPH1 = RPT // ECH


def _sc_gather_reduce(src_tok_pad, esrc_pad, emb):
    mesh = plsc.VectorSubcoreMesh(core_axis_name="c", subcore_axis_name="s")
    cp = pltpu.CompilerParams()
    if "needs_layout_passes" in pltpu.CompilerParams.__dataclass_fields__:
        cp = dataclasses.replace(cp, needs_layout_passes=False)

    @functools.partial(
        pl.kernel,
        compiler_params=cp,
        out_type=[
            jax.ShapeDtypeStruct((NPAD, D), jnp.float32),
            jax.ShapeDtypeStruct((NPAD, D), jnp.float32),
        ],
        mesh=mesh,
        scratch_types=[
            pltpu.VMEM_SHARED((NPAD, D), jnp.float32),
            pltpu.VMEM((TPT * ECH,), jnp.int32),
            pltpu.VMEM((ECH,), jnp.int32),
            pltpu.VMEM((ECH, D), jnp.float32),
            pltpu.VMEM((ECH, D), jnp.float32),
            pltpu.VMEM((CH, D), jnp.float32),
            pltpu.VMEM((CH, D), jnp.float32),
            pltpu.VMEM((CH, D), jnp.float32),
            pltpu.VMEM((CH, D), jnp.float32),
            pltpu.SemaphoreType.DMA,
            pltpu.SemaphoreType.DMA,
            pltpu.SemaphoreType.DMA,
            pltpu.SemaphoreType.DMA,
            pltpu.SemaphoreType.DMA,
            pltpu.SemaphoreType.DMA,
        ],
    )
    def sc_kernel(tok_hbm, esrc_hbm, emb_hbm, total_hbm, xlast_hbm,
                  tab_sh, eidx_v, ridx_v, rows0, rows1, totb0, totb1, xlb0, xlb1,
                  semg0, semg1, semt0, semt1, semx0, semx1):
        cid = lax.axis_index("c")
        sid = lax.axis_index("s")

        pltpu.sync_copy(tok_hbm.at[pl.ds(sid * RPT, RPT)],
                        eidx_v.at[pl.ds(0, RPT)])
        for g in range(GPH1):
            pltpu.async_copy(
                emb_hbm.at[eidx_v.at[pl.ds(g * ECH, ECH)]], rows0, semg0
            ).wait()
            base_row = sid * RPT + g * ECH
            for i8 in range(ECH // 16):
                ridx_v[pl.ds(i8 * 16, 16)] = (
                    base_row + i8 * 16 + lax.iota(jnp.int32, 16))
            pltpu.sync_copy(rows0, tab_sh.at[ridx_v])
        plsc.subcore_barrier()

        wid = sid * 2 + cid
        base_c = wid * TPT
        pltpu.sync_copy(esrc_hbm.at[pl.ds(base_c * ECH, TPT * ECH)], eidx_v)

        def start_gather(x, rows, semg):
            pltpu.async_copy(
                tab_sh.at[eidx_v.at[pl.ds(x * ECH, ECH)]], rows, semg)

        def process(x, rows, semg, totb, xlb, semt, semx, not_first):
            row = (base_c + x) * CH

            @pl.when(not_first)
            def _():
                pltpu.make_async_copy(
                    totb, total_hbm.at[pl.ds(row, CH)], semt).wait()
                pltpu.make_async_copy(
                    xlb, xlast_hbm.at[pl.ds(row, CH)], semx).wait()

            pltpu.make_async_copy(
                tab_sh.at[eidx_v.at[pl.ds(x * ECH, ECH)]], rows, semg).wait()

            def jbody(j, jcarry):
                base = j * K

                def kgbody(kg, accs):
                    kb = base + kg * 8
                    for kk in range(8):
                        for cb in range(D // 16):
                            accs[cb] = accs[cb] + rows[kb + kk, pl.ds(cb * 16, 16)]
                    return accs

                zero = jnp.zeros((16,), jnp.float32)
                accs = lax.fori_loop(0, K // 8, kgbody,
                                     [zero] * (D // 16))
                for cb in range(D // 16):
                    sl = pl.ds(cb * 16, 16)
                    totb[j, sl] = accs[cb]
                    xlb[j, sl] = rows[base + K - 1, sl]
                return jcarry

            lax.fori_loop(0, CH, jbody, 0)
            pltpu.async_copy(totb, total_hbm.at[pl.ds(row, CH)], semt)
            pltpu.async_copy(xlb, xlast_hbm.at[pl.ds(row, CH)], semx)

        start_gather(0, rows0, semg0)

        def body(i, carry):
            a = 2 * i
            start_gather(a + 1, rows1, semg1)
            process(a, rows0, semg0, totb0, xlb0, semt0, semx0,
                    not_first=i >= 1)

            @pl.when(i < TPT // 2 - 1)
            def _():
                start_gather(a + 2, rows0, semg0)

            process(a + 1, rows1, semg1, totb1, xlb1, semt1, semx1,
                    not_first=i >= 1)
            return carry

        lax.fori_loop(0, TPT // 2, body, 0)

        r0 = (base_c + TPT - 2) * CH
        r1 = (base_c + TPT - 1) * CH
        pltpu.make_async_copy(totb0, total_hbm.at[pl.ds(r0, CH)], semt0).wait()
        pltpu.make_async_copy(xlb0, xlast_hbm.at[pl.ds(r0, CH)], semx0).wait()
        pltpu.make_async_copy(totb1, total_hbm.at[pl.ds(r1, CH)], semt1).wait()
        pltpu.make_async_copy(xlb1, xlast_hbm.at[pl.ds(r1, CH)], semx1).wait()

    return sc_kernel(src_tok_pad, esrc_pad, emb)


def _tc_body(x_ref, tot_ref, wih_ref, whh_ref, bih_ref, bhh_ref,
             lng_ref, lnb_ref, wfc_ref, bfc_ref, o_ref):
    x = x_ref[...]
    h = tot_ref[...] - x
    gi = jnp.dot(x, wih_ref[...], preferred_element_type=jnp.float32) + bih_ref[...]
    gh = jnp.dot(h, whh_ref[...], preferred_element_type=jnp.float32) + bhh_ref[...]
    r = jax.nn.sigmoid(gi[:, :D] + gh[:, :D])
    z = jax.nn.sigmoid(gi[:, D:2 * D] + gh[:, D:2 * D])
    n = jnp.tanh(gi[:, 2 * D:] + r * gh[:, 2 * D:])
    ho = (1.0 - z) * n + z * h
    mu = jnp.mean(ho, axis=-1, keepdims=True)
    var = jnp.mean((ho - mu) ** 2, axis=-1, keepdims=True)
    rst = lng_ref[...] * (ho - mu) / jnp.sqrt(var + 1e-5) + lnb_ref[...]
    o_ref[...] = jnp.dot(rst, wfc_ref[...], preferred_element_type=jnp.float32) + bfc_ref[...]


def _tc_head(x, total, W_ihT, W_hhT, b_ih2, b_hh2, ln_g2, ln_b2, W_fcT, b_fc2):
    B = 1024
    grid = (NPAD // B,)
    full = lambda shape: pl.BlockSpec(shape, lambda i: (0, 0))
    return pl.pallas_call(
        _tc_body,
        grid=grid,
        in_specs=[
            pl.BlockSpec((B, D), lambda i: (i, 0)),
            pl.BlockSpec((B, D), lambda i: (i, 0)),
            full((D, 3 * D)),
            full((D, 3 * D)),
            full((1, 3 * D)),
            full((1, 3 * D)),
            full((1, D)),
            full((1, D)),
            full((D, D)),
            full((1, D)),
        ],
        out_specs=pl.BlockSpec((B, D), lambda i: (i, 0)),
        out_shape=jax.ShapeDtypeStruct((NPAD, D), jnp.float32),
    )(x, total, W_ihT, W_hhT, b_ih2, b_hh2, ln_g2, ln_b2, W_fcT, b_fc2)


def kernel(src_token_ids, dst_token_ids, edge_index, emb, W_ih, W_hh,
           b_ih, b_hh, ln_g, ln_b, W_fc, b_fc):
    src_tok_pad = jnp.pad(src_token_ids.astype(jnp.int32), (0, NPAD - N))
    esrc_pad = jnp.pad(edge_index[0].astype(jnp.int32), (0, EPAD - N * K))
    total, xlast = _sc_gather_reduce(src_tok_pad, esrc_pad, emb)

    W_ihT = W_ih.T
    W_hhT = W_hh.T
    b_ih2 = b_ih.reshape(1, 3 * D)
    b_hh2 = b_hh.reshape(1, 3 * D)
    ln_g2 = ln_g.reshape(1, D)
    ln_b2 = ln_b.reshape(1, D)
    W_fcT = jnp.pad(W_fc.T, ((0, 0), (0, D - C)))
    b_fc2 = jnp.pad(b_fc, (0, D - C)).reshape(1, D)

    out_pad = _tc_head(xlast, total, W_ihT, W_hhT, b_ih2, b_hh2,
                       ln_g2, ln_b2, W_fcT, b_fc2)
    return out_pad[:N, :C]


# speedup vs baseline: 5.1184x; 1.1687x over previous
import dataclasses
import functools

import jax
import jax.numpy as jnp
from jax import lax
from jax.experimental import pallas as pl
from jax.experimental.pallas import tpu as pltpu
from jax.experimental.pallas import tpu_sc as plsc

N = 10000
K = 32
D = 128
C = 10

NT = 32
NS = 16
CH = 4
ECH = CH * K
TPT = 80
NCH_PAD = NT * TPT
NPAD = NCH_PAD * CH
EPAD = NCH_PAD * ECH
RPT = NPAD // NS
GPH1 = RPT // ECH


def _sc_gather_reduce(src_tok_pad, esrc_pad, emb):
    mesh = plsc.VectorSubcoreMesh(core_axis_name="c", subcore_axis_name="s")
    cp = pltpu.CompilerParams()
    if "needs_layout_passes" in pltpu.CompilerParams.__dataclass_fields__:
        cp = dataclasses.replace(cp, needs_layout_passes=False)

    @functools.partial(
        pl.kernel,
        compiler_params=cp,
        out_type=[
            jax.ShapeDtypeStruct((NPAD, D), jnp.float32),
            jax.ShapeDtypeStruct((NPAD, D), jnp.float32),
        ],
        mesh=mesh,
        scratch_types=[
            pltpu.VMEM_SHARED((NPAD, D // 2), jnp.int32),
            pltpu.VMEM((TPT * ECH,), jnp.int32),
            pltpu.VMEM((ECH,), jnp.int32),
            pltpu.VMEM((ECH, D), jnp.float32),
            pltpu.VMEM((ECH, D // 2), jnp.int32),
            pltpu.VMEM((ECH, D // 2), jnp.int32),
            pltpu.VMEM((CH, D), jnp.float32),
            pltpu.VMEM((CH, D), jnp.float32),
            pltpu.VMEM((CH, D), jnp.float32),
            pltpu.VMEM((CH, D), jnp.float32),
            pltpu.SemaphoreType.DMA,
            pltpu.SemaphoreType.DMA,
            pltpu.SemaphoreType.DMA,
            pltpu.SemaphoreType.DMA,
            pltpu.SemaphoreType.DMA,
            pltpu.SemaphoreType.DMA,
        ],
    )
    def sc_kernel(tok_hbm, esrc_hbm, emb_hbm, total_hbm, xlast_hbm,
                  tab_sh, eidx_v, ridx_v, rowsf, rows0, rows1, totb0, totb1, xlb0, xlb1,
                  semg0, semg1, semt0, semt1, semx0, semx1):
        cid = lax.axis_index("c")
        sid = lax.axis_index("s")

        pltpu.sync_copy(tok_hbm.at[pl.ds(sid * RPT, RPT)],
                        eidx_v.at[pl.ds(0, RPT)])
        for g in range(GPH1):
            pltpu.async_copy(
                emb_hbm.at[eidx_v.at[pl.ds(g * ECH, ECH)]], rowsf, semg0
            ).wait()

            def cvt(r, ccarry):
                for cp_ in range(D // 32):
                    a = rowsf[r, pl.ds(cp_ * 32, 16)]
                    b = rowsf[r, pl.ds(cp_ * 32 + 16, 16)]
                    packed = plsc.pack(
                        a, b, format=plsc.PackFormat.INTERLEAVED)
                    rows0[r, pl.ds(cp_ * 16, 16)] = plsc.bitcast(
                        packed, jnp.int32)
                return ccarry

            lax.fori_loop(0, ECH, cvt, 0)
            base_row = sid * RPT + g * ECH
            for i8 in range(ECH // 16):
                ridx_v[pl.ds(i8 * 16, 16)] = (
                    base_row + i8 * 16 + lax.iota(jnp.int32, 16))
            pltpu.sync_copy(rows0, tab_sh.at[ridx_v])
        plsc.subcore_barrier()

        wid = sid * 2 + cid
        base_c = wid * TPT
        pltpu.sync_copy(esrc_hbm.at[pl.ds(base_c * ECH, TPT * ECH)], eidx_v)

        def start_gather(x, rows, semg):
            pltpu.async_copy(
                tab_sh.at[eidx_v.at[pl.ds(x * ECH, ECH)]], rows, semg)

        def process(x, rows, semg, totb, xlb, semt, semx, not_first):
            row = (base_c + x) * CH

            @pl.when(not_first)
            def _():
                pltpu.make_async_copy(
                    totb, total_hbm.at[pl.ds(row, CH)], semt).wait()
                pltpu.make_async_copy(
                    xlb, xlast_hbm.at[pl.ds(row, CH)], semx).wait()

            pltpu.make_async_copy(
                tab_sh.at[eidx_v.at[pl.ds(x * ECH, ECH)]], rows, semg).wait()

            def jbody(j, jcarry):
                base = j * K

                def kgbody(kg, accs):
                    kb = base + kg * 8
                    for kk in range(8):
                        for cp_ in range(D // 32):
                            ab = plsc.bitcast(
                                rows[kb + kk, pl.ds(cp_ * 16, 16)],
                                jnp.bfloat16)
                            e0, e1 = plsc.unpack(
                                ab, format=plsc.PackFormat.INTERLEAVED)
                            accs[2 * cp_] = accs[2 * cp_] + e0
                            accs[2 * cp_ + 1] = accs[2 * cp_ + 1] + e1
                    return accs

                zero = jnp.zeros((16,), jnp.float32)
                accs = lax.fori_loop(0, K // 8, kgbody,
                                     [zero] * (D // 16))
                for cp_ in range(D // 32):
                    ab = plsc.bitcast(
                        rows[base + K - 1, pl.ds(cp_ * 16, 16)], jnp.bfloat16)
                    e0, e1 = plsc.unpack(
                        ab, format=plsc.PackFormat.INTERLEAVED)
                    totb[j, pl.ds(cp_ * 32, 16)] = accs[2 * cp_]
                    totb[j, pl.ds(cp_ * 32 + 16, 16)] = accs[2 * cp_ + 1]
                    xlb[j, pl.ds(cp_ * 32, 16)] = e0
                    xlb[j, pl.ds(cp_ * 32 + 16, 16)] = e1
                return jcarry

            lax.fori_loop(0, CH, jbody, 0)
            pltpu.async_copy(totb, total_hbm.at[pl.ds(row, CH)], semt)
            pltpu.async_copy(xlb, xlast_hbm.at[pl.ds(row, CH)], semx)

        start_gather(0, rows0, semg0)

        def body(i, carry):
            a = 2 * i
            start_gather(a + 1, rows1, semg1)
            process(a, rows0, semg0, totb0, xlb0, semt0, semx0,
                    not_first=i >= 1)

            @pl.when(i < TPT // 2 - 1)
            def _():
                start_gather(a + 2, rows0, semg0)

            process(a + 1, rows1, semg1, totb1, xlb1, semt1, semx1,
                    not_first=i >= 1)
            return carry

        lax.fori_loop(0, TPT // 2, body, 0)

        r0 = (base_c + TPT - 2) * CH
        r1 = (base_c + TPT - 1) * CH
        pltpu.make_async_copy(totb0, total_hbm.at[pl.ds(r0, CH)], semt0).wait()
        pltpu.make_async_copy(xlb0, xlast_hbm.at[pl.ds(r0, CH)], semx0).wait()
        pltpu.make_async_copy(totb1, total_hbm.at[pl.ds(r1, CH)], semt1).wait()
        pltpu.make_async_copy(xlb1, xlast_hbm.at[pl.ds(r1, CH)], semx1).wait()

    return sc_kernel(src_tok_pad, esrc_pad, emb)


def _tc_body(x_ref, tot_ref, wih_ref, whh_ref, bih_ref, bhh_ref,
             lng_ref, lnb_ref, wfc_ref, bfc_ref, o_ref):
    x = x_ref[...]
    h = tot_ref[...] - x
    gi = jnp.dot(x, wih_ref[...], preferred_element_type=jnp.float32) + bih_ref[...]
    gh = jnp.dot(h, whh_ref[...], preferred_element_type=jnp.float32) + bhh_ref[...]
    r = jax.nn.sigmoid(gi[:, :D] + gh[:, :D])
    z = jax.nn.sigmoid(gi[:, D:2 * D] + gh[:, D:2 * D])
    n = jnp.tanh(gi[:, 2 * D:] + r * gh[:, 2 * D:])
    ho = (1.0 - z) * n + z * h
    mu = jnp.mean(ho, axis=-1, keepdims=True)
    var = jnp.mean((ho - mu) ** 2, axis=-1, keepdims=True)
    rst = lng_ref[...] * (ho - mu) / jnp.sqrt(var + 1e-5) + lnb_ref[...]
    o_ref[...] = jnp.dot(rst, wfc_ref[...], preferred_element_type=jnp.float32) + bfc_ref[...]


def _tc_head(x, total, W_ihT, W_hhT, b_ih2, b_hh2, ln_g2, ln_b2, W_fcT, b_fc2):
    B = 1024
    grid = (NPAD // B,)
    full = lambda shape: pl.BlockSpec(shape, lambda i: (0, 0))
    return pl.pallas_call(
        _tc_body,
        grid=grid,
        in_specs=[
            pl.BlockSpec((B, D), lambda i: (i, 0)),
            pl.BlockSpec((B, D), lambda i: (i, 0)),
            full((D, 3 * D)),
            full((D, 3 * D)),
            full((1, 3 * D)),
            full((1, 3 * D)),
            full((1, D)),
            full((1, D)),
            full((D, D)),
            full((1, D)),
        ],
        out_specs=pl.BlockSpec((B, D), lambda i: (i, 0)),
        out_shape=jax.ShapeDtypeStruct((NPAD, D), jnp.float32),
    )(x, total, W_ihT, W_hhT, b_ih2, b_hh2, ln_g2, ln_b2, W_fcT, b_fc2)


def kernel(src_token_ids, dst_token_ids, edge_index, emb, W_ih, W_hh,
           b_ih, b_hh, ln_g, ln_b, W_fc, b_fc):
    src_tok_pad = jnp.pad(src_token_ids.astype(jnp.int32), (0, NPAD - N))
    esrc_pad = jnp.pad(edge_index[0].astype(jnp.int32), (0, EPAD - N * K))
    total, xlast = _sc_gather_reduce(src_tok_pad, esrc_pad, emb)

    W_ihT = W_ih.T
    W_hhT = W_hh.T
    b_ih2 = b_ih.reshape(1, 3 * D)
    b_hh2 = b_hh.reshape(1, 3 * D)
    ln_g2 = ln_g.reshape(1, D)
    ln_b2 = ln_b.reshape(1, D)
    W_fcT = jnp.pad(W_fc.T, ((0, 0), (0, D - C)))
    b_fc2 = jnp.pad(b_fc, (0, D - C)).reshape(1, D)

    out_pad = _tc_head(xlast, total, W_ihT, W_hhT, b_ih2, b_hh2,
                       ln_g2, ln_b2, W_fcT, b_fc2)
    return out_pad[:N, :C]
